# baseline (speedup 1.0000x reference)
import functools

import jax
import jax.numpy as jnp
from jax import lax
from jax.experimental import pallas as pl
from jax.experimental.pallas import tpu as pltpu
from jax.experimental.pallas import tpu_sc as plsc

D = 64
NC, NS = 2, 16
NW = NC * NS
BB = 128
CH = 2
PHASES = 4


def _comb_body(minute_ref, hour_ref, weekday_ref, duration_ref, out_ref):
    hm = (jnp.broadcast_to(hour_ref[:][:, None, :], (24, 4, D))
          + jnp.broadcast_to(minute_ref[:][None, :, :], (24, 4, D))).reshape(96, D)
    row = lax.broadcasted_iota(jnp.int32, (7, D), 0) == pl.program_id(0)
    w = jnp.sum(jnp.where(row, weekday_ref[:], 0.0), axis=0)
    out_ref[0] = (hm[:, None, :] + duration_ref[:][None, :, :]
                  + w[None, None, :])


def _build_comb(minute_W, hour_W, weekday_W, duration_W):
    out = pl.pallas_call(
        _comb_body,
        grid=(7,),
        in_specs=[
            pl.BlockSpec((4, D), lambda w: (0, 0)),
            pl.BlockSpec((24, D), lambda w: (0, 0)),
            pl.BlockSpec((7, D), lambda w: (0, 0)),
            pl.BlockSpec((96, D), lambda w: (0, 0)),
        ],
        out_specs=pl.BlockSpec((1, 96, 96, D), lambda w: (w, 0, 0, 0)),
        out_shape=jax.ShapeDtypeStruct((7, 96, 96, D), jnp.float32),
    )(minute_W, hour_W, weekday_W, duration_W)
    return out.reshape(7 * 96 * 96, D)


def _cidx_body(t_ref, w_ref, d_ref, out_ref):
    out_ref[...] = (w_ref[...] * 96 + t_ref[...]) * 96 + d_ref[...]


def _build_cidx(time, weekday, duration):
    B, L = time.shape
    blk = 512
    return pl.pallas_call(
        _cidx_body,
        grid=(B // blk,),
        in_specs=[pl.BlockSpec((blk, L), lambda i: (i, 0))] * 3,
        out_specs=pl.BlockSpec((blk, L), lambda i: (i, 0)),
        out_shape=jax.ShapeDtypeStruct((B, L), jnp.int32),
    )(time, weekday, duration)


def _make_sc_lookup(L, B):
    pl_l = L // PHASES
    nch = pl_l // CH
    mesh = plsc.VectorSubcoreMesh(core_axis_name="c", subcore_axis_name="s")

    @functools.partial(
        pl.kernel,
        mesh=mesh,
        out_type=jax.ShapeDtypeStruct((L, D, B), jnp.float32),
        scratch_types=[
            pltpu.VMEM((pl_l, BB), jnp.int32),
            pltpu.VMEM((pl_l, BB), jnp.int32),
            pltpu.VMEM((2, CH, BB, D), jnp.float32),
            pltpu.VMEM((2, CH, BB, D), jnp.float32),
            pltpu.VMEM((2, CH, D, BB), jnp.float32),
            pltpu.SemaphoreType.DMA,
            pltpu.SemaphoreType.DMA,
        ],
        compiler_params=pltpu.CompilerParams(use_tc_tiling_on_sc=False,
                                             needs_layout_passes=False),
    )
    def sc_lookup(srcT_h, cidxT_h, comb_h, loc_h, out_h,
                  sblk, cblk, rowsA, rowsB, outT, semg, semw):
        cid = lax.axis_index("c")
        sid = lax.axis_index("s")
        wid = sid * NC + cid
        b0 = wid * BB

        def fire_gathers(k, p):
            for j in range(CH):
                pltpu.async_copy(loc_h.at[sblk.at[k * CH + j]],
                                 rowsA.at[p, j], semg)
                pltpu.async_copy(comb_h.at[cblk.at[k * CH + j]],
                                 rowsB.at[p, j], semg)

        def drain_gathers():
            for _ in range(2 * CH):
                pltpu.make_async_copy(loc_h.at[pl.ds(0, BB)],
                                      rowsA.at[0, 0], semg).wait()

        def drain_write(lb):
            pltpu.make_async_copy(
                outT.at[0],
                out_h.at[pl.ds(lb, CH), pl.ds(0, D), pl.ds(b0, BB)],
                semw).wait()

        @pl.loop(0, PHASES)
        def _phase(phase):
            lb = phase * pl_l
            pltpu.sync_copy(srcT_h.at[pl.ds(lb, pl_l), pl.ds(b0, BB)], sblk)
            pltpu.sync_copy(cidxT_h.at[pl.ds(lb, pl_l), pl.ds(b0, BB)], cblk)

            fire_gathers(0, 0)

            @pl.loop(0, nch)
            def _chunk(k):
                p = lax.rem(k, 2)

                @pl.when(k < nch - 1)
                def _():
                    fire_gathers(k + 1, lax.rem(k + 1, 2))

                drain_gathers()

                @pl.when(k >= 2)
                def _():
                    drain_write(lb)

                pvec = jnp.broadcast_to(p, (16,))
                for j in range(CH):
                    jvec = jnp.full((16,), j, jnp.int32)

                    @pl.loop(0, BB // 16)
                    def _g(g):
                        bvec = lax.iota(jnp.int32, 16) + (16 * g)

                        @pl.loop(0, D, unroll=8)
                        def _d(d):
                            dvec = jnp.broadcast_to(d, (16,))
                            val = (plsc.load_gather(rowsA, [pvec, jvec, bvec, dvec])
                                   + plsc.load_gather(rowsB, [pvec, jvec, bvec, dvec]))
                            outT[p, j, d, pl.ds(16 * g, 16)] = val

                pltpu.async_copy(
                    outT.at[p],
                    out_h.at[pl.ds(lb + k * CH, CH), pl.ds(0, D),
                             pl.ds(b0, BB)],
                    semw)

            drain_write(lb)
            drain_write(lb)

    return sc_lookup


def kernel(src, time, weekday, duration, emb_loc_W, minute_W, hour_W,
           weekday_W, duration_W):
    B, L = src.shape
    comb = _build_comb(minute_W, hour_W, weekday_W, duration_W)
    cidx = _build_cidx(time.astype(jnp.int32), weekday.astype(jnp.int32),
                       duration.astype(jnp.int32))
    srcT = jnp.transpose(src.astype(jnp.int32))
    cidxT = jnp.transpose(cidx)
    out = _make_sc_lookup(L, B)(srcT, cidxT, comb, emb_loc_W)
    return jnp.transpose(out, (2, 0, 1))


# speedup vs baseline: 2.1260x; 2.1260x over previous
import functools

import jax
import jax.numpy as jnp
from jax import lax
from jax.experimental import pallas as pl
from jax.experimental.pallas import tpu as pltpu
from jax.experimental.pallas import tpu_sc as plsc

D = 64
NC, NS = 2, 16
NW = NC * NS
BB = 128
CH = 2
PHASES = 4


def _comb_body(minute_ref, hour_ref, weekday_ref, duration_ref, out_ref):
    hm = (jnp.broadcast_to(hour_ref[:][:, None, :], (24, 4, D))
          + jnp.broadcast_to(minute_ref[:][None, :, :], (24, 4, D))).reshape(96, D)
    row = lax.broadcasted_iota(jnp.int32, (7, D), 0) == pl.program_id(0)
    w = jnp.sum(jnp.where(row, weekday_ref[:], 0.0), axis=0)
    out_ref[0] = (hm[:, None, :] + duration_ref[:][None, :, :]
                  + w[None, None, :])


def _build_comb(minute_W, hour_W, weekday_W, duration_W):
    out = pl.pallas_call(
        _comb_body,
        grid=(7,),
        in_specs=[
            pl.BlockSpec((4, D), lambda w: (0, 0)),
            pl.BlockSpec((24, D), lambda w: (0, 0)),
            pl.BlockSpec((7, D), lambda w: (0, 0)),
            pl.BlockSpec((96, D), lambda w: (0, 0)),
        ],
        out_specs=pl.BlockSpec((1, 96, 96, D), lambda w: (w, 0, 0, 0)),
        out_shape=jax.ShapeDtypeStruct((7, 96, 96, D), jnp.float32),
    )(minute_W, hour_W, weekday_W, duration_W)
    return out.reshape(7 * 96 * 96, D)


def _cidx_body(t_ref, w_ref, d_ref, out_ref):
    out_ref[...] = (w_ref[...] * 96 + t_ref[...]) * 96 + d_ref[...]


def _build_cidx(time, weekday, duration):
    B, L = time.shape
    blk = 512
    return pl.pallas_call(
        _cidx_body,
        grid=(B // blk,),
        in_specs=[pl.BlockSpec((blk, L), lambda i: (i, 0))] * 3,
        out_specs=pl.BlockSpec((blk, L), lambda i: (i, 0)),
        out_shape=jax.ShapeDtypeStruct((B, L), jnp.int32),
    )(time, weekday, duration)


def _make_sc_lookup(L, B):
    pl_l = L // PHASES
    nch = pl_l // CH
    mesh = plsc.VectorSubcoreMesh(core_axis_name="c", subcore_axis_name="s")

    @functools.partial(
        pl.kernel,
        mesh=mesh,
        out_type=jax.ShapeDtypeStruct((L, D, B), jnp.float32),
        scratch_types=[
            pltpu.VMEM((pl_l, BB), jnp.int32),
            pltpu.VMEM((pl_l, BB), jnp.int32),
            pltpu.VMEM((2, CH, BB, D), jnp.float32),
            pltpu.VMEM((2, CH, BB, D), jnp.float32),
            pltpu.VMEM((2, CH, D, BB), jnp.float32),
            pltpu.SemaphoreType.DMA,
            pltpu.SemaphoreType.DMA,
        ],
        compiler_params=pltpu.CompilerParams(use_tc_tiling_on_sc=False,
                                             needs_layout_passes=False),
    )
    def sc_lookup(srcT_h, cidxT_h, comb_h, loc_h, out_h,
                  sblk, cblk, rowsA, rowsB, outT, semg, semw):
        cid = lax.axis_index("c")
        sid = lax.axis_index("s")
        wid = sid * NC + cid
        b0 = wid * BB

        def fire_gathers(k, p):
            for j in range(CH):
                pltpu.async_copy(loc_h.at[sblk.at[k * CH + j]],
                                 rowsA.at[p, j], semg)
                pltpu.async_copy(comb_h.at[cblk.at[k * CH + j]],
                                 rowsB.at[p, j], semg)

        def drain_gathers():
            for _ in range(2 * CH):
                pltpu.make_async_copy(loc_h.at[pl.ds(0, BB)],
                                      rowsA.at[0, 0], semg).wait()

        def drain_write(lb):
            pltpu.make_async_copy(
                outT.at[0],
                out_h.at[pl.ds(lb, CH), pl.ds(0, D), pl.ds(b0, BB)],
                semw).wait()

        @pl.loop(0, PHASES)
        def _phase(phase):
            lb = phase * pl_l
            pltpu.sync_copy(srcT_h.at[pl.ds(lb, pl_l), pl.ds(b0, BB)], sblk)
            pltpu.sync_copy(cidxT_h.at[pl.ds(lb, pl_l), pl.ds(b0, BB)], cblk)

            fire_gathers(0, 0)

            @pl.loop(0, nch)
            def _chunk(k):
                p = lax.rem(k, 2)

                @pl.when(k < nch - 1)
                def _():
                    fire_gathers(k + 1, lax.rem(k + 1, 2))

                drain_gathers()

                @pl.when(k >= 2)
                def _():
                    drain_write(lb)

                pvec = jnp.broadcast_to(p, (16,))
                iot = lax.iota(jnp.int32, 16)
                for j in range(CH):
                    jvec = jnp.full((16,), j, jnp.int32)

                    @pl.loop(0, BB // 16)
                    def _g(g):
                        bvec = iot + (16 * g)

                        @pl.loop(0, D // 16)
                        def _q(q):
                            @pl.loop(0, 16, unroll=4)
                            def _k(k):
                                dvec = (16 * q) + ((iot + k) & 15)
                                val = (plsc.load_gather(
                                            rowsA, [pvec, jvec, bvec, dvec])
                                       + plsc.load_gather(
                                            rowsB, [pvec, jvec, bvec, dvec]))
                                plsc.store_scatter(
                                    outT, [pvec, jvec, dvec, bvec], val)

                pltpu.async_copy(
                    outT.at[p],
                    out_h.at[pl.ds(lb + k * CH, CH), pl.ds(0, D),
                             pl.ds(b0, BB)],
                    semw)

            drain_write(lb)
            drain_write(lb)

    return sc_lookup


def kernel(src, time, weekday, duration, emb_loc_W, minute_W, hour_W,
           weekday_W, duration_W):
    B, L = src.shape
    comb = _build_comb(minute_W, hour_W, weekday_W, duration_W)
    cidx = _build_cidx(time.astype(jnp.int32), weekday.astype(jnp.int32),
                       duration.astype(jnp.int32))
    srcT = jnp.transpose(src.astype(jnp.int32))
    cidxT = jnp.transpose(cidx)
    out = _make_sc_lookup(L, B)(srcT, cidxT, comb, emb_loc_W)
    return jnp.transpose(out, (2, 0, 1))


# speedup vs baseline: 2.1402x; 1.0067x over previous
import functools

import jax
import jax.numpy as jnp
from jax import lax
from jax.experimental import pallas as pl
from jax.experimental.pallas import tpu as pltpu
from jax.experimental.pallas import tpu_sc as plsc

D = 64
NC, NS = 2, 16
NW = NC * NS
BB = 512
NQ = BB // 128
LG = 8
PHASES = 2


def _comb_body(minute_ref, hour_ref, weekday_ref, duration_ref, out_ref):
    hm = (jnp.broadcast_to(hour_ref[:][:, None, :], (24, 4, D))
          + jnp.broadcast_to(minute_ref[:][None, :, :], (24, 4, D))).reshape(96, D)
    row = lax.broadcasted_iota(jnp.int32, (7, D), 0) == pl.program_id(0)
    w = jnp.sum(jnp.where(row, weekday_ref[:], 0.0), axis=0)
    out_ref[0] = (hm[:, None, :] + duration_ref[:][None, :, :]
                  + w[None, None, :])


def _build_comb(minute_W, hour_W, weekday_W, duration_W):
    out = pl.pallas_call(
        _comb_body,
        grid=(7,),
        in_specs=[
            pl.BlockSpec((4, D), lambda w: (0, 0)),
            pl.BlockSpec((24, D), lambda w: (0, 0)),
            pl.BlockSpec((7, D), lambda w: (0, 0)),
            pl.BlockSpec((96, D), lambda w: (0, 0)),
        ],
        out_specs=pl.BlockSpec((1, 96, 96, D), lambda w: (w, 0, 0, 0)),
        out_shape=jax.ShapeDtypeStruct((7, 96, 96, D), jnp.float32),
    )(minute_W, hour_W, weekday_W, duration_W)
    return out.reshape(7 * 96 * 96, D)


def _cidx_body(t_ref, w_ref, d_ref, out_ref):
    out_ref[...] = (w_ref[...] * 96 + t_ref[...]) * 96 + d_ref[...]


def _build_cidx(time, weekday, duration):
    B, L = time.shape
    blk = 512
    return pl.pallas_call(
        _cidx_body,
        grid=(B // blk,),
        in_specs=[pl.BlockSpec((blk, L), lambda i: (i, 0))] * 3,
        out_specs=pl.BlockSpec((blk, L), lambda i: (i, 0)),
        out_shape=jax.ShapeDtypeStruct((B, L), jnp.int32),
    )(time, weekday, duration)


def _make_sc_lookup(L, B):
    lpw = L // (NW // LG)
    pl_l = lpw // PHASES
    nst = pl_l * NQ
    mesh = plsc.VectorSubcoreMesh(core_axis_name="c", subcore_axis_name="s")

    @functools.partial(
        pl.kernel,
        mesh=mesh,
        out_type=jax.ShapeDtypeStruct((L, D, B), jnp.float32),
        scratch_types=[
            pltpu.VMEM((pl_l, BB), jnp.int32),
            pltpu.VMEM((pl_l, BB), jnp.int32),
            pltpu.VMEM((2, 128, D), jnp.float32),
            pltpu.VMEM((2, 128, D), jnp.float32),
            pltpu.VMEM((2, D, BB), jnp.float32),
            pltpu.SemaphoreType.DMA,
            pltpu.SemaphoreType.DMA,
        ],
        compiler_params=pltpu.CompilerParams(use_tc_tiling_on_sc=False,
                                             needs_layout_passes=False),
    )
    def sc_lookup(srcT_h, cidxT_h, comb_h, loc_h, out_h,
                  sblk, cblk, rowsA, rowsB, outT, semg, semw):
        cid = lax.axis_index("c")
        sid = lax.axis_index("s")
        wid = sid * NC + cid
        b0 = lax.rem(wid, LG) * BB
        l00 = (wid // LG) * lpw

        def fire_gathers(s, p):
            l = s >> 2
            q = s & 3
            pltpu.async_copy(loc_h.at[sblk.at[l, pl.ds(128 * q, 128)]],
                             rowsA.at[p], semg)
            pltpu.async_copy(comb_h.at[cblk.at[l, pl.ds(128 * q, 128)]],
                             rowsB.at[p], semg)

        def drain_gathers():
            for _ in range(2):
                pltpu.make_async_copy(loc_h.at[pl.ds(0, 128)],
                                      rowsA.at[0], semg).wait()

        def drain_write():
            pltpu.make_async_copy(
                outT.at[0],
                out_h.at[0, pl.ds(0, D), pl.ds(0, BB)],
                semw).wait()

        @pl.loop(0, PHASES)
        def _phase(phase):
            lb = l00 + phase * pl_l
            pltpu.sync_copy(srcT_h.at[pl.ds(lb, pl_l), pl.ds(b0, BB)], sblk)
            pltpu.sync_copy(cidxT_h.at[pl.ds(lb, pl_l), pl.ds(b0, BB)], cblk)

            fire_gathers(0, 0)

            @pl.loop(0, nst)
            def _step(s):
                l = s >> 2
                q = s & 3
                p = s & 1

                @pl.when(s < nst - 1)
                def _():
                    fire_gathers(s + 1, (s + 1) & 1)

                drain_gathers()

                @pl.when((q == 0) & (l >= 2))
                def _():
                    drain_write()

                pvec = jnp.broadcast_to(p, (16,))
                lpar = jnp.broadcast_to(l & 1, (16,))
                iot = lax.iota(jnp.int32, 16)

                @pl.loop(0, 8)
                def _g(g):
                    bvec = iot + (16 * g)
                    ovec = bvec + (128 * q)

                    @pl.loop(0, D // 16)
                    def _dq(dq):
                        @pl.loop(0, 16, unroll=4)
                        def _k(k):
                            dvec = (16 * dq) + ((iot + k) & 15)
                            val = (plsc.load_gather(
                                        rowsA, [pvec, bvec, dvec])
                                   + plsc.load_gather(
                                        rowsB, [pvec, bvec, dvec]))
                            plsc.store_scatter(
                                outT, [lpar, dvec, ovec], val)

                @pl.when(q == 3)
                def _():
                    pltpu.async_copy(
                        outT.at[l & 1],
                        out_h.at[lb + l, pl.ds(0, D), pl.ds(b0, BB)],
                        semw)

            drain_write()
            drain_write()

    return sc_lookup


def kernel(src, time, weekday, duration, emb_loc_W, minute_W, hour_W,
           weekday_W, duration_W):
    B, L = src.shape
    comb = _build_comb(minute_W, hour_W, weekday_W, duration_W)
    cidx = _build_cidx(time.astype(jnp.int32), weekday.astype(jnp.int32),
                       duration.astype(jnp.int32))
    srcT = jnp.transpose(src.astype(jnp.int32))
    cidxT = jnp.transpose(cidx)
    out = _make_sc_lookup(L, B)(srcT, cidxT, comb, emb_loc_W)
    return jnp.transpose(out, (2, 0, 1))
